# baseline (speedup 1.0000x reference)
import functools

import jax
import jax.numpy as jnp
from jax import lax
from jax.experimental import pallas as pl
from jax.experimental.pallas import tpu as pltpu
from jax.experimental.pallas import tpu_sc as plsc

N = 10000
E = 320000
D = 128
H = 64
G = 100
NPG = 100
EPS = 1e-3

NUM_CORES = 2
NUM_SUBCORES = 16
CH = 128
NROWS = E // CH
NCF = 80
NCL = NROWS - 31 * NCF
ZR = N // NUM_SUBCORES
ZRC = ZR // 5


NBUF = 2


def _pipe(h_hbm, a_hbm, src2_v, dst2_v, rows, gsem, ssem,
          acc_sh, base, nchunk):
    pltpu.sync_copy(a_hbm.at[0, pl.ds(base, nchunk)],
                    src2_v.at[pl.ds(0, nchunk)])
    pltpu.sync_copy(a_hbm.at[1, pl.ds(base, nchunk)],
                    dst2_v.at[pl.ds(0, nchunk)])
    for b in range(NBUF):
        pltpu.async_copy(h_hbm.at[src2_v.at[b]], rows[b], gsem[b])

    def body(k, carry):
        j0 = NBUF * k
        for b in range(NBUF):
            j = j0 + b
            pltpu.make_async_copy(h_hbm.at[src2_v.at[j]], rows[b],
                                  gsem[b]).wait()
            pltpu.async_copy(rows[b], acc_sh.at[dst2_v.at[j]], ssem[b],
                             add=True)
        for b in range(NBUF):
            j = j0 + b

            @pl.when(j + NBUF < nchunk)
            def _():
                pltpu.make_async_copy(rows[b], acc_sh.at[dst2_v.at[j]],
                                      ssem[b]).wait()
                pltpu.async_copy(h_hbm.at[src2_v.at[j + NBUF]], rows[b],
                                 gsem[b])
        return carry

    lax.fori_loop(0, nchunk // NBUF, body, 0)

    for b in range(NBUF):
        j = nchunk - NBUF + b
        pltpu.make_async_copy(rows[b], acc_sh.at[dst2_v.at[j]],
                              ssem[b]).wait()


def _edge_agg_body(h_hbm, a_hbm, zero_hbm, out_hbm,
                   src2_v, dst2_v, r0, r1, wb_v, acc_sh, h_sh,
                   g0, g1, s0, s1):
    rows = (r0, r1)
    gsem = (g0, g1)
    ssem = (s0, s1)
    c = lax.axis_index("c")
    s = lax.axis_index("s")

    pltpu.sync_copy(zero_hbm, wb_v)
    for t in range(5):
        sl = pl.ds(s * ZR + t * ZRC, ZRC)
        pltpu.sync_copy(wb_v, acc_sh.at[sl])
        pltpu.sync_copy(h_hbm.at[sl], h_sh.at[sl])
    plsc.subcore_barrier()

    @pl.when(c == 0)
    def _():
        _pipe(h_sh, a_hbm, src2_v, dst2_v, rows, gsem, ssem,
              acc_sh, s * NCF, NCF)

    @pl.when(jnp.logical_and(c == 1, s < NUM_SUBCORES - 1))
    def _():
        _pipe(h_sh, a_hbm, src2_v, dst2_v, rows, gsem, ssem,
              acc_sh, (NUM_SUBCORES + s) * NCF, NCF)

    @pl.when(jnp.logical_and(c == 1, s == NUM_SUBCORES - 1))
    def _():
        _pipe(h_sh, a_hbm, src2_v, dst2_v, rows, gsem, ssem,
              acc_sh, 31 * NCF, NCL)

    plsc.subcore_barrier()

    for t in range(5):
        wsl = pl.ds(s * ZR + t * ZRC, ZRC)
        pltpu.sync_copy(acc_sh.at[wsl], wb_v)
        pltpu.sync_copy(wb_v, out_hbm.at[c, wsl])


@functools.cache
def _build_edge_agg():
    return pl.kernel(
        _edge_agg_body,
        mesh=plsc.VectorSubcoreMesh(core_axis_name="c", subcore_axis_name="s"),
        out_type=jax.ShapeDtypeStruct((NUM_CORES, N, H), jnp.float32),
        scratch_types=[
            pltpu.VMEM((NCF, CH), jnp.int32),
            pltpu.VMEM((NCF, CH), jnp.int32),
            pltpu.VMEM((CH, H), jnp.float32),
            pltpu.VMEM((CH, H), jnp.float32),
            pltpu.VMEM((ZRC, H), jnp.float32),
            pltpu.VMEM_SHARED((N, H), jnp.float32),
            pltpu.VMEM_SHARED((N, H), jnp.float32),
        ] + [pltpu.SemaphoreType.DMA] * 4,
        compiler_params=pltpu.CompilerParams(use_tc_tiling_on_sc=False),
    )


def _edge_agg(h, a3, zeros):
    return _build_edge_agg()(h, a3, zeros)


def _elu(x):
    return jnp.where(x > 0, x, jnp.exp(jnp.minimum(x, 0.0)) - 1.0)


def _dense1_body(x_ref, w_ref, c_ref, o_ref):
    o_ref[...] = jnp.dot(x_ref[...], w_ref[...],
                         preferred_element_type=jnp.float32) + c_ref[...]


def _dense2_body(p_ref, w_ref, c_ref, o_ref):
    t = _elu(p_ref[0] + p_ref[1])
    o_ref[...] = jnp.dot(t, w_ref[...],
                         preferred_element_type=jnp.float32) + c_ref[...]


def _combine_body(p_ref, o_ref):
    o_ref[...] = _elu(p_ref[0] + p_ref[1])


def _head_body(g_ref, w1_ref, b1_ref, w2_ref, b2_ref, w3_ref, b3_ref, o_ref):
    t = jax.nn.relu(jnp.dot(g_ref[...], w1_ref[...],
                            preferred_element_type=jnp.float32) + b1_ref[...])
    t = jax.nn.relu(jnp.dot(t, w2_ref[...],
                            preferred_element_type=jnp.float32) + b2_ref[...])
    o_ref[...] = jax.nn.sigmoid(jnp.dot(t, w3_ref[...],
                                        preferred_element_type=jnp.float32)
                                + b3_ref[...])


def _dense1(x, w, c):
    return pl.pallas_call(
        _dense1_body,
        out_shape=jax.ShapeDtypeStruct((N, H), jnp.float32),
    )(x, w, c)


def _dense2(p, w, c):
    return pl.pallas_call(
        _dense2_body,
        out_shape=jax.ShapeDtypeStruct((N, H), jnp.float32),
    )(p, w, c)


def _combine(p):
    return pl.pallas_call(
        _combine_body,
        out_shape=jax.ShapeDtypeStruct((N, H), jnp.float32),
    )(p)


def _head(g, w1, b1, w2, b2, w3, b3):
    return pl.pallas_call(
        _head_body,
        out_shape=jax.ShapeDtypeStruct((G, 1), jnp.float32),
    )(g, w1, b1, w2, b2, w3, b3)


def kernel(x, a, i, W1, b1, g1, be1, m1, v1, W2, b2, g2, be2, m2, v2,
           Wd1, bd1, Wd2, bd2, Wd3, bd3):
    inv1 = g1 / jnp.sqrt(v1 + EPS)
    W1f = W1 * inv1[None, :]
    c1 = ((b1 - m1) * inv1 + be1)[None, :]
    inv2 = g2 / jnp.sqrt(v2 + EPS)
    W2f = W2 * inv2[None, :]
    c2 = ((b2 - m2) * inv2 + be2)[None, :]

    a3 = a.reshape(2, NROWS, CH)
    zeros = jnp.zeros((ZRC, H), jnp.float32)

    h1 = _dense1(x, W1f, c1)
    p1 = _edge_agg(h1, a3, zeros)
    h2 = _dense2(p1, W2f, c2)
    p2 = _edge_agg(h2, a3, zeros)
    e2 = _combine(p2)
    g = e2.reshape(G, NPG * H)
    return _head(g, Wd1, bd1[None, :], Wd2, bd2[None, :], Wd3, bd3[None, :])


# speedup vs baseline: 1.2426x; 1.2426x over previous
import functools

import jax
import jax.numpy as jnp
from jax import lax
from jax.experimental import pallas as pl
from jax.experimental.pallas import tpu as pltpu
from jax.experimental.pallas import tpu_sc as plsc

N = 10000
E = 320000
D = 128
H = 64
G = 100
NPG = 100
EPS = 1e-3

NUM_CORES = 2
NUM_SUBCORES = 16
CH = 128
NROWS = E // CH
NCF = 80
NCL = NROWS - 31 * NCF
ZR = N // NUM_SUBCORES
ZRC = ZR // 5


NBUF = 4


def _pipe(h_hbm, a_hbm, src2_v, dst2_v, rows, gsem, ssem,
          acc_sh, base, nchunk):
    pltpu.sync_copy(a_hbm.at[0, pl.ds(base, nchunk)],
                    src2_v.at[pl.ds(0, nchunk)])
    pltpu.sync_copy(a_hbm.at[1, pl.ds(base, nchunk)],
                    dst2_v.at[pl.ds(0, nchunk)])
    for b in range(NBUF):
        pltpu.async_copy(h_hbm.at[src2_v.at[b]], rows[b], gsem[b])

    def body(k, carry):
        j0 = NBUF * k
        for b in range(NBUF):
            j = j0 + b
            pltpu.make_async_copy(h_hbm.at[src2_v.at[j]], rows[b],
                                  gsem[b]).wait()
            pltpu.async_copy(rows[b], acc_sh.at[dst2_v.at[j]], ssem[b],
                             add=True)
        for b in range(NBUF):
            j = j0 + b

            @pl.when(j + NBUF < nchunk)
            def _():
                pltpu.make_async_copy(rows[b], acc_sh.at[dst2_v.at[j]],
                                      ssem[b]).wait()
                pltpu.async_copy(h_hbm.at[src2_v.at[j + NBUF]], rows[b],
                                 gsem[b])
        return carry

    lax.fori_loop(0, nchunk // NBUF, body, 0)

    for b in range(NBUF):
        j = nchunk - NBUF + b
        pltpu.make_async_copy(rows[b], acc_sh.at[dst2_v.at[j]],
                              ssem[b]).wait()


def _edge_agg_body(h_hbm, a_hbm, zero_hbm, out_hbm,
                   src2_v, dst2_v, r0, r1, r2, r3, wb_v, acc_sh,
                   g0, g1, g2, g3, s0, s1, s2, s3):
    rows = (r0, r1, r2, r3)
    gsem = (g0, g1, g2, g3)
    ssem = (s0, s1, s2, s3)
    c = lax.axis_index("c")
    s = lax.axis_index("s")

    pltpu.sync_copy(zero_hbm, wb_v)
    for t in range(5):
        pltpu.sync_copy(wb_v, acc_sh.at[pl.ds(s * ZR + t * ZRC, ZRC)])
    plsc.subcore_barrier()

    @pl.when(c == 0)
    def _():
        _pipe(h_hbm, a_hbm, src2_v, dst2_v, rows, gsem, ssem,
              acc_sh, s * NCF, NCF)

    @pl.when(jnp.logical_and(c == 1, s < NUM_SUBCORES - 1))
    def _():
        _pipe(h_hbm, a_hbm, src2_v, dst2_v, rows, gsem, ssem,
              acc_sh, (NUM_SUBCORES + s) * NCF, NCF)

    @pl.when(jnp.logical_and(c == 1, s == NUM_SUBCORES - 1))
    def _():
        _pipe(h_hbm, a_hbm, src2_v, dst2_v, rows, gsem, ssem,
              acc_sh, 31 * NCF, NCL)

    plsc.subcore_barrier()

    for t in range(5):
        wsl = pl.ds(s * ZR + t * ZRC, ZRC)
        pltpu.sync_copy(acc_sh.at[wsl], wb_v)
        pltpu.sync_copy(wb_v, out_hbm.at[c, wsl])


@functools.cache
def _build_edge_agg():
    return pl.kernel(
        _edge_agg_body,
        mesh=plsc.VectorSubcoreMesh(core_axis_name="c", subcore_axis_name="s"),
        out_type=jax.ShapeDtypeStruct((NUM_CORES, N, H), jnp.float32),
        scratch_types=[
            pltpu.VMEM((NCF, CH), jnp.int32),
            pltpu.VMEM((NCF, CH), jnp.int32),
            pltpu.VMEM((CH, H), jnp.float32),
            pltpu.VMEM((CH, H), jnp.float32),
            pltpu.VMEM((CH, H), jnp.float32),
            pltpu.VMEM((CH, H), jnp.float32),
            pltpu.VMEM((ZRC, H), jnp.float32),
            pltpu.VMEM_SHARED((N, H), jnp.float32),
        ] + [pltpu.SemaphoreType.DMA] * 8,
        compiler_params=pltpu.CompilerParams(use_tc_tiling_on_sc=False),
    )


def _edge_agg(h, a3, zeros):
    return _build_edge_agg()(h, a3, zeros)


def _elu(x):
    return jnp.where(x > 0, x, jnp.exp(jnp.minimum(x, 0.0)) - 1.0)


def _dense1_body(x_ref, w_ref, c_ref, o_ref):
    o_ref[...] = jnp.dot(x_ref[...], w_ref[...],
                         preferred_element_type=jnp.float32) + c_ref[...]


def _dense2_body(p_ref, w_ref, c_ref, o_ref):
    t = _elu(p_ref[0] + p_ref[1])
    o_ref[...] = jnp.dot(t, w_ref[...],
                         preferred_element_type=jnp.float32) + c_ref[...]


def _combine_body(p_ref, o_ref):
    o_ref[...] = _elu(p_ref[0] + p_ref[1])


def _head_body(g_ref, w1_ref, b1_ref, w2_ref, b2_ref, w3_ref, b3_ref, o_ref):
    t = jax.nn.relu(jnp.dot(g_ref[...], w1_ref[...],
                            preferred_element_type=jnp.float32) + b1_ref[...])
    t = jax.nn.relu(jnp.dot(t, w2_ref[...],
                            preferred_element_type=jnp.float32) + b2_ref[...])
    o_ref[...] = jax.nn.sigmoid(jnp.dot(t, w3_ref[...],
                                        preferred_element_type=jnp.float32)
                                + b3_ref[...])


def _dense1(x, w, c):
    return pl.pallas_call(
        _dense1_body,
        out_shape=jax.ShapeDtypeStruct((N, H), jnp.float32),
    )(x, w, c)


def _dense2(p, w, c):
    return pl.pallas_call(
        _dense2_body,
        out_shape=jax.ShapeDtypeStruct((N, H), jnp.float32),
    )(p, w, c)


def _combine(p):
    return pl.pallas_call(
        _combine_body,
        out_shape=jax.ShapeDtypeStruct((N, H), jnp.float32),
    )(p)


def _head(g, w1, b1, w2, b2, w3, b3):
    return pl.pallas_call(
        _head_body,
        out_shape=jax.ShapeDtypeStruct((G, 1), jnp.float32),
    )(g, w1, b1, w2, b2, w3, b3)


def kernel(x, a, i, W1, b1, g1, be1, m1, v1, W2, b2, g2, be2, m2, v2,
           Wd1, bd1, Wd2, bd2, Wd3, bd3):
    inv1 = g1 / jnp.sqrt(v1 + EPS)
    W1f = W1 * inv1[None, :]
    c1 = ((b1 - m1) * inv1 + be1)[None, :]
    inv2 = g2 / jnp.sqrt(v2 + EPS)
    W2f = W2 * inv2[None, :]
    c2 = ((b2 - m2) * inv2 + be2)[None, :]

    a3 = a.reshape(2, NROWS, CH)
    zeros = jnp.zeros((ZRC, H), jnp.float32)

    h1 = _dense1(x, W1f, c1)
    p1 = _edge_agg(h1, a3, zeros)
    h2 = _dense2(p1, W2f, c2)
    p2 = _edge_agg(h2, a3, zeros)
    e2 = _combine(p2)
    g = e2.reshape(G, NPG * H)
    return _head(g, Wd1, bd1[None, :], Wd2, bd2[None, :], Wd3, bd3[None, :])
